# baseline (speedup 1.0000x reference)
import functools

import jax
import jax.numpy as jnp
from jax import lax
from jax.experimental import pallas as pl
from jax.experimental.pallas import tpu as pltpu
from jax.experimental.pallas import tpu_sc as plsc

N = 10000
E = 320000
P = 100000
F = 128
OUT = 2

NC = 2
NS = 16
NW = NC * NS

NP = 10240
_APT = NP // NS

CH_E = 80
CH_G = 128
CH_S = 80

E_CHT = E // (NW * CH_E)
G_CHT = 75
GP_TOT = NW * G_CHT * CH_G
S_PT = 6400
SP_TOT = S_PT * NS

_Q = 8
_D = 4


@functools.cache
def _mesh():
    return plsc.VectorSubcoreMesh(core_axis_name="c", subcore_axis_name="s",
                                  num_cores=NC, num_subcores=NS)


def _al(v):
    return pl.multiple_of(v, 8)


def _pipeline(iters, gather_spec, scat_spec, idx_spec, n_gat, n_scat,
              scat_add=True):
    lead = _Q - n_scat
    assert n_gat + n_scat <= _D and lead >= n_gat

    def start(descs, **kw):
        for d in descs:
            d.start(**kw)

    def wait(descs):
        for d in descs:
            d.wait()

    for i in range(lead):
        start(idx_spec(i, i % _Q))
    for i in range(n_gat):
        wait(idx_spec(i, i % _Q))
        start(gather_spec(i, i % _Q, i % _D))

    def step(i, j):
        q, b = j % _Q, j % _D
        wait(gather_spec(i, q, b))
        if scat_add:
            start(scat_spec(i, q, b), add=True)
        else:
            start(scat_spec(i, q, b))

        @pl.when(i >= n_scat)
        def _():
            wait(scat_spec(i - n_scat, (j - n_scat) % _Q, (j - n_scat) % _D))

        @pl.when(i + lead < iters)
        def _():
            start(idx_spec(i + lead, (j + lead) % _Q))

        @pl.when(i + n_gat < iters)
        def _():
            wait(idx_spec(i + n_gat, (j + n_gat) % _Q))
            start(gather_spec(i + n_gat, (j + n_gat) % _Q, (j + n_gat) % _D))

    def body(k, carry):
        i0 = _Q * k
        for j in range(_Q):
            step(i0 + j, j)
        return carry

    nb = iters // _Q
    lax.fori_loop(0, nb, body, 0)
    for j in range(iters - _Q * nb):
        step(jnp.int32(_Q * nb + j), j)
    for i in range(max(iters - n_scat, 0), iters):
        wait(scat_spec(jnp.int32(i), i % _Q, i % _D))


def _make_seg_body(with_deg):
    def _seg_body(table, src, dst, zeros, *rest):
        if with_deg:
            (ones, parts, partsd, idxs, idxd, rows, onesv, acc, accd,
             sem_i, sem_g, sem_s, sem_d) = rest
        else:
            parts, idxs, idxd, rows, acc, sem_i, sem_g, sem_s = rest
        cid = lax.axis_index("c")
        sid = lax.axis_index("s")
        arow = _al(sid * _APT)
        pltpu.sync_copy(zeros.at[pl.ds(arow, _APT), :],
                        acc.at[pl.ds(arow, _APT), :])
        if with_deg:
            pltpu.sync_copy(zeros.at[pl.ds(arow, _APT), 0:8],
                            accd.at[pl.ds(arow, _APT), :])
            pltpu.sync_copy(ones, onesv)
        plsc.subcore_barrier()
        wid = sid * NC + cid
        iters = E_CHT
        base = _al(wid * (E_CHT * CH_E))

        def idx_spec(i, q):
            return [
                pltpu.make_async_copy(src.at[pl.ds(base + _al(i * CH_E), CH_E)],
                                      idxs.at[q], sem_i[q]),
                pltpu.make_async_copy(dst.at[pl.ds(base + _al(i * CH_E), CH_E)],
                                      idxd.at[q], sem_i[q]),
            ]

        def gather_spec(i, q, b):
            return [pltpu.make_async_copy(table.at[idxs.at[q]], rows.at[b],
                                          sem_g[b])]

        def scat_spec(i, q, b):
            ds = [pltpu.make_async_copy(rows.at[b], acc.at[idxd.at[q]],
                                        sem_s[b])]
            if with_deg:
                ds.append(pltpu.make_async_copy(onesv, accd.at[idxd.at[q]],
                                                sem_d[b]))
            return ds

        _pipeline(iters, gather_spec, scat_spec, idx_spec, n_gat=2, n_scat=2)
        plsc.subcore_barrier()
        pltpu.sync_copy(acc.at[pl.ds(arow, _APT), :],
                        parts.at[cid, pl.ds(arow, _APT), :])
        if with_deg:
            pltpu.sync_copy(accd.at[pl.ds(arow, _APT), :],
                            partsd.at[cid, pl.ds(arow, _APT), :])

    return _seg_body


@functools.cache
def _make_seg(with_deg):
    dma = pltpu.SemaphoreType.DMA
    out_type = [jax.ShapeDtypeStruct((NC, NP, F), jnp.float32)]
    scratch = [
        pltpu.VMEM((_Q, CH_E), jnp.int32),
        pltpu.VMEM((_Q, CH_E), jnp.int32),
        pltpu.VMEM((_D, CH_E, F), jnp.float32),
    ]
    if with_deg:
        out_type.append(jax.ShapeDtypeStruct((NC, NP, 8), jnp.float32))
        scratch += [pltpu.VMEM((CH_E, 8), jnp.float32),
                    pltpu.VMEM_SHARED((NP, F), jnp.float32),
                    pltpu.VMEM_SHARED((NP, 8), jnp.float32),
                    [dma] * _Q, [dma] * _D, [dma] * _D, [dma] * _D]
    else:
        scratch += [pltpu.VMEM_SHARED((NP, F), jnp.float32),
                    [dma] * _Q, [dma] * _D, [dma] * _D]
    return pl.kernel(
        _make_seg_body(with_deg),
        out_type=tuple(out_type) if with_deg else out_type[0],
        mesh=_mesh(),
        compiler_params=pltpu.CompilerParams(use_tc_tiling_on_sc=False),
        scratch_types=scratch,
    )


def _gather_body(table, idx, out, idx4, rows, sem_i, sem_g, sem_w):
    cid = lax.axis_index("c")
    sid = lax.axis_index("s")
    wid = sid * NC + cid
    iters = G_CHT
    base = _al(wid * (G_CHT * CH_G))

    def idx_spec(i, q):
        return [pltpu.make_async_copy(idx.at[pl.ds(base + _al(i * CH_G), CH_G)],
                                      idx4.at[q], sem_i[q])]

    def gather_spec(i, q, b):
        return [pltpu.make_async_copy(table.at[idx4.at[q]], rows.at[b],
                                      sem_g[b])]

    def write_spec(i, q, b):
        return [pltpu.make_async_copy(
            rows.at[b], out.at[pl.ds(base + _al(i * CH_G), CH_G), :],
            sem_w[b])]

    _pipeline(iters, gather_spec, write_spec, idx_spec, n_gat=3, n_scat=1,
              scat_add=False)


@functools.cache
def _make_gather():
    dma = pltpu.SemaphoreType.DMA
    return pl.kernel(
        _gather_body,
        out_type=jax.ShapeDtypeStruct((GP_TOT, F), jnp.bfloat16),
        mesh=_mesh(),
        compiler_params=pltpu.CompilerParams(use_tc_tiling_on_sc=False),
        scratch_types=[
            pltpu.VMEM((_Q, CH_G), jnp.int32),
            pltpu.VMEM((_D, CH_G, F), jnp.bfloat16),
            [dma] * _Q,
            [dma] * _D,
            [dma] * _D,
        ],
    )


def _scatter_body(logp, place, zeros, out, idx4, valv, acc,
                  sem_i, sem_v, sem_s):
    cid = lax.axis_index("c")
    sid = lax.axis_index("s")

    @pl.when(cid == 0)
    def _():
        arow = _al(sid * _APT)
        pltpu.sync_copy(zeros.at[pl.ds(arow, _APT), :],
                        acc.at[pl.ds(arow, _APT), :])
        plsc.subcore_barrier()
        base = _al(sid * S_PT)
        iters = S_PT // CH_S

        def idx_spec(i, q):
            return [pltpu.make_async_copy(
                place.at[pl.ds(base + _al(i * CH_S), CH_S)],
                idx4.at[q], sem_i[q])]

        def val_spec(i, q, b):
            return [pltpu.make_async_copy(
                logp.at[pl.ds(base + _al(i * CH_S), CH_S), :],
                valv.at[b], sem_v[b])]

        def scat_spec(i, q, b):
            return [pltpu.make_async_copy(valv.at[b], acc.at[idx4.at[q]],
                                          sem_s[b])]

        _pipeline(iters, val_spec, scat_spec, idx_spec, n_gat=2, n_scat=2)
        plsc.subcore_barrier()
        pltpu.sync_copy(acc.at[pl.ds(arow, _APT), :],
                        out.at[pl.ds(arow, _APT), :])


@functools.cache
def _make_scatter():
    dma = pltpu.SemaphoreType.DMA
    return pl.kernel(
        _scatter_body,
        out_type=jax.ShapeDtypeStruct((NP, 8), jnp.float32),
        mesh=_mesh(),
        compiler_params=pltpu.CompilerParams(use_tc_tiling_on_sc=False),
        scratch_types=[
            pltpu.VMEM((_Q, CH_S), jnp.int32),
            pltpu.VMEM((_D, CH_S, 8), jnp.float32),
            pltpu.VMEM_SHARED((NP, 8), jnp.float32),
            [dma] * _Q,
            [dma] * _D,
            [dma] * _D,
        ],
    )


_R = 400


def _enc1_body(x_ref, a_ref, d_ref, w_ref, b_ref, h_ref, inv_ref):
    asum = a_ref[0] + a_ref[1]
    deg = jnp.maximum(d_ref[0][:, 0:1] + d_ref[1][:, 0:1], 1.0)
    inv = 1.0 / deg
    agg = asum * inv
    pre = jnp.dot(x_ref[...] + agg, w_ref[...],
                  preferred_element_type=jnp.float32) + b_ref[...]
    h_ref[...] = jnp.maximum(pre, 0.0)
    inv_ref[...] = jnp.broadcast_to(inv, (_R, 8))


_enc1 = pl.pallas_call(
    _enc1_body,
    grid=(N // _R,),
    in_specs=[
        pl.BlockSpec((_R, F), lambda i: (i, 0)),
        pl.BlockSpec((NC, _R, F), lambda i: (0, i, 0)),
        pl.BlockSpec((NC, _R, 8), lambda i: (0, i, 0)),
        pl.BlockSpec((F, F), lambda i: (0, 0)),
        pl.BlockSpec((1, F), lambda i: (0, 0)),
    ],
    out_specs=[
        pl.BlockSpec((_R, F), lambda i: (i, 0)),
        pl.BlockSpec((_R, 8), lambda i: (i, 0)),
    ],
    out_shape=[
        jax.ShapeDtypeStruct((N, F), jnp.float32),
        jax.ShapeDtypeStruct((N, 8), jnp.float32),
    ],
)


def _enc2_body(h_ref, a_ref, inv_ref, w_ref, b_ref, h2_ref):
    asum = a_ref[0] + a_ref[1]
    agg = asum * inv_ref[:, 0:1]
    pre = jnp.dot(h_ref[...] + agg, w_ref[...],
                  preferred_element_type=jnp.float32) + b_ref[...]
    h2_ref[...] = jnp.maximum(pre, 0.0).astype(jnp.bfloat16)


_enc2 = pl.pallas_call(
    _enc2_body,
    grid=(N // _R,),
    in_specs=[
        pl.BlockSpec((_R, F), lambda i: (i, 0)),
        pl.BlockSpec((NC, _R, F), lambda i: (0, i, 0)),
        pl.BlockSpec((_R, 8), lambda i: (i, 0)),
        pl.BlockSpec((F, F), lambda i: (0, 0)),
        pl.BlockSpec((1, F), lambda i: (0, 0)),
    ],
    out_specs=pl.BlockSpec((_R, F), lambda i: (i, 0)),
    out_shape=jax.ShapeDtypeStruct((N, F), jnp.bfloat16),
)


def _dec_body(f0_ref, f1_ref, f2_ref, w1_ref, b1_ref, w2_ref, b2_ref, o_ref):
    w = w1_ref[...]
    hid = (jnp.dot(f0_ref[...], w[0:128], preferred_element_type=jnp.float32)
           + jnp.dot(f1_ref[...], w[128:256],
                     preferred_element_type=jnp.float32)
           + jnp.dot(f2_ref[...], w[256:384],
                     preferred_element_type=jnp.float32)
           + b1_ref[...])
    hid = jnp.maximum(hid, 0.0)
    logits = jnp.dot(hid, w2_ref[...],
                     preferred_element_type=jnp.float32) + b2_ref[...]
    l0 = logits[:, 0:1]
    l1 = logits[:, 1:2]
    m = jnp.maximum(l0, l1)
    lse = m + jnp.log(jnp.exp(l0 - m) + jnp.exp(l1 - m))
    o_ref[...] = logits - lse


_PB = P // _R
_dec = pl.pallas_call(
    _dec_body,
    grid=(_PB,),
    in_specs=[
        pl.BlockSpec((_R, F), lambda i: (i, 0)),
        pl.BlockSpec((_R, F), lambda i: (i + _PB, 0)),
        pl.BlockSpec((_R, F), lambda i: (i + 2 * _PB, 0)),
        pl.BlockSpec((3 * F, 3 * F), lambda i: (0, 0)),
        pl.BlockSpec((1, 3 * F), lambda i: (0, 0)),
        pl.BlockSpec((3 * F, 8), lambda i: (0, 0)),
        pl.BlockSpec((1, 8), lambda i: (0, 0)),
    ],
    out_specs=pl.BlockSpec((_R, 8), lambda i: (i, 0)),
    out_shape=jax.ShapeDtypeStruct((SP_TOT, 8), jnp.float32),
)


def kernel(x, edge_index, original, y, nodes, variants,
           W1, b1, W2, b2, Wd1, bd1, Wd2, bd2):
    f32 = jnp.float32
    i32 = jnp.int32
    src = edge_index[0]
    dst = edge_index[1]
    zeros = jnp.zeros((NP, F), f32)
    ones8 = jnp.ones((CH_E, 8), f32)

    parts1, deg1 = _make_seg(True)(x, src, dst, zeros, ones8)
    h, inv8 = _enc1(x, parts1, deg1, W1, b1.reshape(1, F))
    parts2 = _make_seg(False)(h, src, dst, zeros)
    h2 = _enc2(h, parts2, inv8, W2, b2.reshape(1, F))

    gpad = jnp.arange(GP_TOT - 3 * P, dtype=i32) % N
    gidx = jnp.concatenate([variants.reshape(3 * P), gpad])
    feats = _make_gather()(h2, gidx)

    Wd2p = jnp.concatenate([Wd2, jnp.zeros((3 * F, 8 - OUT), f32)], axis=1)
    bd2p = jnp.concatenate([bd2, jnp.zeros((8 - OUT,), f32)]).reshape(1, 8)
    logp8 = _dec(feats, feats, feats, Wd1.astype(jnp.bfloat16),
                 bd1.reshape(1, 3 * F), Wd2p, bd2p)

    spad = N + jnp.arange(SP_TOT - P, dtype=i32) % (NP - N)
    placep = jnp.concatenate([variants[0], spad])
    pred8 = _make_scatter()(logp8, placep, jnp.zeros((NP, 8), f32))
    return pred8[:N, :OUT]


# speedup vs baseline: 1.4897x; 1.4897x over previous
import functools

import jax
import jax.numpy as jnp
from jax import lax
from jax.experimental import pallas as pl
from jax.experimental.pallas import tpu as pltpu
from jax.experimental.pallas import tpu_sc as plsc

N = 10000
E = 320000
P = 100000
F = 128
OUT = 2

NC = 2
NS = 16
NW = NC * NS

NP = 10240
_APT = NP // NS

CH_E = 80
CH_G = 128
CH_S = 80

E_CHT = E // (NW * CH_E)
G_CHT = 75
GP_TOT = NW * G_CHT * CH_G
S_PT = 6400
SP_TOT = S_PT * NS

_Q = 8
_D = 4


@functools.cache
def _mesh():
    return plsc.VectorSubcoreMesh(core_axis_name="c", subcore_axis_name="s",
                                  num_cores=NC, num_subcores=NS)


def _al(v):
    return pl.multiple_of(v, 8)


def _pipeline(iters, gather_spec, scat_spec, idx_spec, n_gat, n_scat,
              scat_add=True):
    lead = _Q - n_scat
    assert n_gat + n_scat <= _D and lead >= n_gat

    def start(descs, **kw):
        for d in descs:
            d.start(**kw)

    def wait(descs):
        for d in descs:
            d.wait()

    for i in range(lead):
        start(idx_spec(i, i % _Q))
    for i in range(n_gat):
        wait(idx_spec(i, i % _Q))
        start(gather_spec(i, i % _Q, i % _D))

    def step(i, j):
        q, b = j % _Q, j % _D
        wait(gather_spec(i, q, b))
        if scat_add:
            start(scat_spec(i, q, b), add=True)
        else:
            start(scat_spec(i, q, b))

        @pl.when(i >= n_scat)
        def _():
            wait(scat_spec(i - n_scat, (j - n_scat) % _Q, (j - n_scat) % _D))

        @pl.when(i + lead < iters)
        def _():
            start(idx_spec(i + lead, (j + lead) % _Q))

        @pl.when(i + n_gat < iters)
        def _():
            wait(idx_spec(i + n_gat, (j + n_gat) % _Q))
            start(gather_spec(i + n_gat, (j + n_gat) % _Q, (j + n_gat) % _D))

    def body(k, carry):
        i0 = _Q * k
        for j in range(_Q):
            step(i0 + j, j)
        return carry

    nb = iters // _Q
    lax.fori_loop(0, nb, body, 0)
    for j in range(iters - _Q * nb):
        step(jnp.int32(_Q * nb + j), j)
    for i in range(max(iters - n_scat, 0), iters):
        wait(scat_spec(jnp.int32(i), i % _Q, i % _D))


def _make_seg_body(with_deg):
    def _seg_body(table, src, dst, zeros, *rest):
        if with_deg:
            (ones, parts, partsd, idxs, idxd, rows, onesv, acc, accd,
             sem_i, sem_g, sem_s, sem_d) = rest
        else:
            parts, idxs, idxd, rows, acc, sem_i, sem_g, sem_s = rest
        cid = lax.axis_index("c")
        sid = lax.axis_index("s")
        arow = _al(sid * _APT)
        pltpu.sync_copy(zeros.at[pl.ds(arow, _APT), :],
                        acc.at[pl.ds(arow, _APT), :])
        if with_deg:
            pltpu.sync_copy(zeros.at[pl.ds(arow, _APT), 0:8],
                            accd.at[pl.ds(arow, _APT), :])
            pltpu.sync_copy(ones, onesv)
        plsc.subcore_barrier()
        wid = sid * NC + cid
        iters = E_CHT
        base = _al(wid * (E_CHT * CH_E))

        def idx_spec(i, q):
            return [
                pltpu.make_async_copy(src.at[pl.ds(base + _al(i * CH_E), CH_E)],
                                      idxs.at[q], sem_i[q]),
                pltpu.make_async_copy(dst.at[pl.ds(base + _al(i * CH_E), CH_E)],
                                      idxd.at[q], sem_i[q]),
            ]

        def gather_spec(i, q, b):
            return [pltpu.make_async_copy(table.at[idxs.at[q]], rows.at[b],
                                          sem_g[b])]

        def scat_spec(i, q, b):
            ds = [pltpu.make_async_copy(rows.at[b], acc.at[idxd.at[q]],
                                        sem_s[b])]
            if with_deg:
                ds.append(pltpu.make_async_copy(onesv, accd.at[idxd.at[q]],
                                                sem_d[b]))
            return ds

        _pipeline(iters, gather_spec, scat_spec, idx_spec, n_gat=2, n_scat=2)
        plsc.subcore_barrier()
        pltpu.sync_copy(acc.at[pl.ds(arow, _APT), :],
                        parts.at[cid, pl.ds(arow, _APT), :])
        if with_deg:
            pltpu.sync_copy(accd.at[pl.ds(arow, _APT), :],
                            partsd.at[cid, pl.ds(arow, _APT), :])

    return _seg_body


@functools.cache
def _make_seg(with_deg):
    dma = pltpu.SemaphoreType.DMA
    out_type = [jax.ShapeDtypeStruct((NC, NP, F), jnp.float32)]
    scratch = [
        pltpu.VMEM((_Q, CH_E), jnp.int32),
        pltpu.VMEM((_Q, CH_E), jnp.int32),
        pltpu.VMEM((_D, CH_E, F), jnp.float32),
    ]
    if with_deg:
        out_type.append(jax.ShapeDtypeStruct((NC, NP, 8), jnp.float32))
        scratch += [pltpu.VMEM((CH_E, 8), jnp.float32),
                    pltpu.VMEM_SHARED((NP, F), jnp.float32),
                    pltpu.VMEM_SHARED((NP, 8), jnp.float32),
                    [dma] * _Q, [dma] * _D, [dma] * _D, [dma] * _D]
    else:
        scratch += [pltpu.VMEM_SHARED((NP, F), jnp.float32),
                    [dma] * _Q, [dma] * _D, [dma] * _D]
    return pl.kernel(
        _make_seg_body(with_deg),
        out_type=tuple(out_type) if with_deg else out_type[0],
        mesh=_mesh(),
        compiler_params=pltpu.CompilerParams(use_tc_tiling_on_sc=False),
        scratch_types=scratch,
    )


def _gather_body(table, idx, out, idx4, rows, sem_i, sem_g, sem_w):
    cid = lax.axis_index("c")
    sid = lax.axis_index("s")
    wid = sid * NC + cid
    iters = G_CHT
    base = _al(wid * (G_CHT * CH_G))

    def idx_spec(i, q):
        return [pltpu.make_async_copy(idx.at[pl.ds(base + _al(i * CH_G), CH_G)],
                                      idx4.at[q], sem_i[q])]

    def gather_spec(i, q, b):
        return [pltpu.make_async_copy(table.at[idx4.at[q]], rows.at[b],
                                      sem_g[b])]

    def write_spec(i, q, b):
        return [pltpu.make_async_copy(
            rows.at[b], out.at[pl.ds(base + _al(i * CH_G), CH_G), :],
            sem_w[b])]

    _pipeline(iters, gather_spec, write_spec, idx_spec, n_gat=3, n_scat=1,
              scat_add=False)


@functools.cache
def _make_gather():
    dma = pltpu.SemaphoreType.DMA
    return pl.kernel(
        _gather_body,
        out_type=jax.ShapeDtypeStruct((GP_TOT, F), jnp.float32),
        mesh=_mesh(),
        compiler_params=pltpu.CompilerParams(use_tc_tiling_on_sc=False),
        scratch_types=[
            pltpu.VMEM((_Q, CH_G), jnp.int32),
            pltpu.VMEM((_D, CH_G, F), jnp.float32),
            [dma] * _Q,
            [dma] * _D,
            [dma] * _D,
        ],
    )


def _scatter_body(logp, place, zeros, out, idx4, valv, acc,
                  sem_i, sem_v, sem_s):
    cid = lax.axis_index("c")
    sid = lax.axis_index("s")

    @pl.when(cid == 0)
    def _():
        arow = _al(sid * _APT)
        pltpu.sync_copy(zeros.at[pl.ds(arow, _APT), :],
                        acc.at[pl.ds(arow, _APT), :])
        plsc.subcore_barrier()
        base = _al(sid * S_PT)
        iters = S_PT // CH_S

        def idx_spec(i, q):
            return [pltpu.make_async_copy(
                place.at[pl.ds(base + _al(i * CH_S), CH_S)],
                idx4.at[q], sem_i[q])]

        def val_spec(i, q, b):
            return [pltpu.make_async_copy(
                logp.at[pl.ds(base + _al(i * CH_S), CH_S), :],
                valv.at[b], sem_v[b])]

        def scat_spec(i, q, b):
            return [pltpu.make_async_copy(valv.at[b], acc.at[idx4.at[q]],
                                          sem_s[b])]

        _pipeline(iters, val_spec, scat_spec, idx_spec, n_gat=2, n_scat=2)
        plsc.subcore_barrier()
        pltpu.sync_copy(acc.at[pl.ds(arow, _APT), :],
                        out.at[pl.ds(arow, _APT), :])


@functools.cache
def _make_scatter():
    dma = pltpu.SemaphoreType.DMA
    return pl.kernel(
        _scatter_body,
        out_type=jax.ShapeDtypeStruct((NP, 8), jnp.float32),
        mesh=_mesh(),
        compiler_params=pltpu.CompilerParams(use_tc_tiling_on_sc=False),
        scratch_types=[
            pltpu.VMEM((_Q, CH_S), jnp.int32),
            pltpu.VMEM((_D, CH_S, 8), jnp.float32),
            pltpu.VMEM_SHARED((NP, 8), jnp.float32),
            [dma] * _Q,
            [dma] * _D,
            [dma] * _D,
        ],
    )


_R = 400


def _enc1_body(x_ref, a_ref, d_ref, w_ref, b_ref, h_ref, inv_ref):
    asum = a_ref[0] + a_ref[1]
    deg = jnp.maximum(d_ref[0][:, 0:1] + d_ref[1][:, 0:1], 1.0)
    inv = 1.0 / deg
    agg = asum * inv
    pre = jnp.dot(x_ref[...] + agg, w_ref[...],
                  preferred_element_type=jnp.float32) + b_ref[...]
    h_ref[...] = jnp.maximum(pre, 0.0)
    inv_ref[...] = jnp.broadcast_to(inv, (_R, 8))


_enc1 = pl.pallas_call(
    _enc1_body,
    grid=(N // _R,),
    in_specs=[
        pl.BlockSpec((_R, F), lambda i: (i, 0)),
        pl.BlockSpec((NC, _R, F), lambda i: (0, i, 0)),
        pl.BlockSpec((NC, _R, 8), lambda i: (0, i, 0)),
        pl.BlockSpec((F, F), lambda i: (0, 0)),
        pl.BlockSpec((1, F), lambda i: (0, 0)),
    ],
    out_specs=[
        pl.BlockSpec((_R, F), lambda i: (i, 0)),
        pl.BlockSpec((_R, 8), lambda i: (i, 0)),
    ],
    out_shape=[
        jax.ShapeDtypeStruct((N, F), jnp.float32),
        jax.ShapeDtypeStruct((N, 8), jnp.float32),
    ],
)


def _enc2_body(h_ref, a_ref, inv_ref, w_ref, b_ref, h2_ref):
    asum = a_ref[0] + a_ref[1]
    agg = asum * inv_ref[:, 0:1]
    pre = jnp.dot(h_ref[...] + agg, w_ref[...],
                  preferred_element_type=jnp.float32) + b_ref[...]
    h2_ref[...] = jnp.maximum(pre, 0.0)


_enc2 = pl.pallas_call(
    _enc2_body,
    grid=(N // _R,),
    in_specs=[
        pl.BlockSpec((_R, F), lambda i: (i, 0)),
        pl.BlockSpec((NC, _R, F), lambda i: (0, i, 0)),
        pl.BlockSpec((_R, 8), lambda i: (i, 0)),
        pl.BlockSpec((F, F), lambda i: (0, 0)),
        pl.BlockSpec((1, F), lambda i: (0, 0)),
    ],
    out_specs=pl.BlockSpec((_R, F), lambda i: (i, 0)),
    out_shape=jax.ShapeDtypeStruct((N, F), jnp.float32),
)


def _dec_body(f0_ref, f1_ref, f2_ref, w1_ref, b1_ref, w2_ref, b2_ref, o_ref):
    w = w1_ref[...]
    hid = (jnp.dot(f0_ref[...], w[0:128], preferred_element_type=jnp.float32)
           + jnp.dot(f1_ref[...], w[128:256],
                     preferred_element_type=jnp.float32)
           + jnp.dot(f2_ref[...], w[256:384],
                     preferred_element_type=jnp.float32)
           + b1_ref[...])
    hid = jnp.maximum(hid, 0.0)
    logits = jnp.dot(hid, w2_ref[...],
                     preferred_element_type=jnp.float32) + b2_ref[...]
    l0 = logits[:, 0:1]
    l1 = logits[:, 1:2]
    m = jnp.maximum(l0, l1)
    lse = m + jnp.log(jnp.exp(l0 - m) + jnp.exp(l1 - m))
    o_ref[...] = logits - lse


_RD = 1000
_PB = P // _RD
_dec = pl.pallas_call(
    _dec_body,
    grid=(_PB,),
    in_specs=[
        pl.BlockSpec((_RD, F), lambda i: (i, 0)),
        pl.BlockSpec((_RD, F), lambda i: (i + _PB, 0)),
        pl.BlockSpec((_RD, F), lambda i: (i + 2 * _PB, 0)),
        pl.BlockSpec((3 * F, 3 * F), lambda i: (0, 0)),
        pl.BlockSpec((1, 3 * F), lambda i: (0, 0)),
        pl.BlockSpec((3 * F, 8), lambda i: (0, 0)),
        pl.BlockSpec((1, 8), lambda i: (0, 0)),
    ],
    out_specs=pl.BlockSpec((_RD, 8), lambda i: (i, 0)),
    out_shape=jax.ShapeDtypeStruct((SP_TOT, 8), jnp.float32),
)


def kernel(x, edge_index, original, y, nodes, variants,
           W1, b1, W2, b2, Wd1, bd1, Wd2, bd2):
    f32 = jnp.float32
    i32 = jnp.int32
    src = edge_index[0]
    dst = edge_index[1]
    zeros = jnp.zeros((NP, F), f32)
    ones8 = jnp.ones((CH_E, 8), f32)

    parts1, deg1 = _make_seg(True)(x, src, dst, zeros, ones8)
    h, inv8 = _enc1(x, parts1, deg1, W1, b1.reshape(1, F))
    parts2 = _make_seg(False)(h, src, dst, zeros)
    h2 = _enc2(h, parts2, inv8, W2, b2.reshape(1, F))

    gpad = jnp.arange(GP_TOT - 3 * P, dtype=i32) % N
    gidx = jnp.concatenate([variants.reshape(3 * P), gpad])
    feats = _make_gather()(h2, gidx)

    Wd2p = jnp.concatenate([Wd2, jnp.zeros((3 * F, 8 - OUT), f32)], axis=1)
    bd2p = jnp.concatenate([bd2, jnp.zeros((8 - OUT,), f32)]).reshape(1, 8)
    logp8 = _dec(feats, feats, feats, Wd1,
                 bd1.reshape(1, 3 * F), Wd2p, bd2p)

    spad = N + jnp.arange(SP_TOT - P, dtype=i32) % (NP - N)
    placep = jnp.concatenate([variants[0], spad])
    pred8 = _make_scatter()(logp8, placep, jnp.zeros((NP, 8), f32))
    return pred8[:N, :OUT]


# speedup vs baseline: 1.5508x; 1.0410x over previous
import functools

import jax
import jax.numpy as jnp
from jax import lax
from jax.experimental import pallas as pl
from jax.experimental.pallas import tpu as pltpu
from jax.experimental.pallas import tpu_sc as plsc

N = 10000
E = 320000
P = 100000
F = 128
OUT = 2

NC = 2
NS = 16
NW = NC * NS

NP = 10240
_APT = NP // NS

CH_E = 80
CH_G = 128
CH_S = 80

E_CHT = E // (NW * CH_E)
G_CHT = 75
GP_TOT = NW * G_CHT * CH_G
S_PT = 6400
SP_TOT = S_PT * NS

_Q = 8
_D = 4


@functools.cache
def _mesh():
    return plsc.VectorSubcoreMesh(core_axis_name="c", subcore_axis_name="s",
                                  num_cores=NC, num_subcores=NS)


def _al(v):
    return pl.multiple_of(v, 8)


def _pipeline(iters, gather_spec, scat_spec, idx_spec, n_gat, n_scat,
              scat_add=True):
    lead = _Q - n_scat
    assert n_gat + n_scat <= _D and lead >= n_gat

    def start(descs, **kw):
        for d in descs:
            d.start(**kw)

    def wait(descs):
        for d in descs:
            d.wait()

    for i in range(lead):
        start(idx_spec(i, i % _Q))
    for i in range(n_gat):
        wait(idx_spec(i, i % _Q))
        start(gather_spec(i, i % _Q, i % _D))

    def step(i, j):
        q, b = j % _Q, j % _D
        wait(gather_spec(i, q, b))
        if scat_add:
            start(scat_spec(i, q, b), add=True)
        else:
            start(scat_spec(i, q, b))

        @pl.when(i >= n_scat)
        def _():
            wait(scat_spec(i - n_scat, (j - n_scat) % _Q, (j - n_scat) % _D))

        @pl.when(i + lead < iters)
        def _():
            start(idx_spec(i + lead, (j + lead) % _Q))

        @pl.when(i + n_gat < iters)
        def _():
            wait(idx_spec(i + n_gat, (j + n_gat) % _Q))
            start(gather_spec(i + n_gat, (j + n_gat) % _Q, (j + n_gat) % _D))

    def body(k, carry):
        i0 = _Q * k
        for j in range(_Q):
            step(i0 + j, j)
        return carry

    nb = iters // _Q
    lax.fori_loop(0, nb, body, 0)
    for j in range(iters - _Q * nb):
        step(jnp.int32(_Q * nb + j), j)
    for i in range(max(iters - n_scat, 0), iters):
        wait(scat_spec(jnp.int32(i), i % _Q, i % _D))


def _make_seg_body(with_deg):
    def _seg_body(table, src, dst, zeros, *rest):
        if with_deg:
            (ones, parts, partsd, idxs, idxd, rows, onesv, acc, accd,
             sem_i, sem_g, sem_s, sem_d) = rest
        else:
            parts, idxs, idxd, rows, acc, sem_i, sem_g, sem_s = rest
        cid = lax.axis_index("c")
        sid = lax.axis_index("s")
        arow = _al(sid * _APT)
        pltpu.sync_copy(zeros.at[pl.ds(arow, _APT), :],
                        acc.at[pl.ds(arow, _APT), :])
        if with_deg:
            pltpu.sync_copy(zeros.at[pl.ds(arow, _APT), 0:8],
                            accd.at[pl.ds(arow, _APT), :])
            pltpu.sync_copy(ones, onesv)
        plsc.subcore_barrier()
        wid = sid * NC + cid
        iters = E_CHT
        base = _al(wid * (E_CHT * CH_E))

        def idx_spec(i, q):
            return [
                pltpu.make_async_copy(src.at[pl.ds(base + _al(i * CH_E), CH_E)],
                                      idxs.at[q], sem_i[q]),
                pltpu.make_async_copy(dst.at[pl.ds(base + _al(i * CH_E), CH_E)],
                                      idxd.at[q], sem_i[q]),
            ]

        def gather_spec(i, q, b):
            return [pltpu.make_async_copy(table.at[idxs.at[q]], rows.at[b],
                                          sem_g[b])]

        def scat_spec(i, q, b):
            ds = [pltpu.make_async_copy(rows.at[b], acc.at[idxd.at[q]],
                                        sem_s[b])]
            if with_deg:
                ds.append(pltpu.make_async_copy(onesv, accd.at[idxd.at[q]],
                                                sem_d[b]))
            return ds

        _pipeline(iters, gather_spec, scat_spec, idx_spec, n_gat=2, n_scat=2)
        plsc.subcore_barrier()
        pltpu.sync_copy(acc.at[pl.ds(arow, _APT), :],
                        parts.at[cid, pl.ds(arow, _APT), :])
        if with_deg:
            pltpu.sync_copy(accd.at[pl.ds(arow, _APT), :],
                            partsd.at[cid, pl.ds(arow, _APT), :])

    return _seg_body


@functools.cache
def _make_seg(with_deg):
    dma = pltpu.SemaphoreType.DMA
    out_type = [jax.ShapeDtypeStruct((NC, NP, F), jnp.float32)]
    scratch = [
        pltpu.VMEM((_Q, CH_E), jnp.int32),
        pltpu.VMEM((_Q, CH_E), jnp.int32),
        pltpu.VMEM((_D, CH_E, F), jnp.float32),
    ]
    if with_deg:
        out_type.append(jax.ShapeDtypeStruct((NC, NP, 8), jnp.float32))
        scratch += [pltpu.VMEM((CH_E, 8), jnp.float32),
                    pltpu.VMEM_SHARED((NP, F), jnp.float32),
                    pltpu.VMEM_SHARED((NP, 8), jnp.float32),
                    [dma] * _Q, [dma] * _D, [dma] * _D, [dma] * _D]
    else:
        scratch += [pltpu.VMEM_SHARED((NP, F), jnp.float32),
                    [dma] * _Q, [dma] * _D, [dma] * _D]
    return pl.kernel(
        _make_seg_body(with_deg),
        out_type=tuple(out_type) if with_deg else out_type[0],
        mesh=_mesh(),
        compiler_params=pltpu.CompilerParams(use_tc_tiling_on_sc=False),
        scratch_types=scratch,
    )


def _gather_body(table, idx, out, idx4, rows, sem_i, sem_g, sem_w):
    cid = lax.axis_index("c")
    sid = lax.axis_index("s")
    wid = sid * NC + cid
    iters = G_CHT
    base = _al(wid * (G_CHT * CH_G))

    def idx_spec(i, q):
        return [pltpu.make_async_copy(idx.at[pl.ds(base + _al(i * CH_G), CH_G)],
                                      idx4.at[q], sem_i[q])]

    def gather_spec(i, q, b):
        return [pltpu.make_async_copy(table.at[idx4.at[q]], rows.at[b],
                                      sem_g[b])]

    def write_spec(i, q, b):
        return [pltpu.make_async_copy(
            rows.at[b], out.at[pl.ds(base + _al(i * CH_G), CH_G), :],
            sem_w[b])]

    _pipeline(iters, gather_spec, write_spec, idx_spec, n_gat=3, n_scat=1,
              scat_add=False)


@functools.cache
def _make_gather():
    dma = pltpu.SemaphoreType.DMA
    return pl.kernel(
        _gather_body,
        out_type=jax.ShapeDtypeStruct((GP_TOT, F), jnp.float32),
        mesh=_mesh(),
        compiler_params=pltpu.CompilerParams(use_tc_tiling_on_sc=False),
        scratch_types=[
            pltpu.VMEM((_Q, CH_G), jnp.int32),
            pltpu.VMEM((_D, CH_G, F), jnp.float32),
            [dma] * _Q,
            [dma] * _D,
            [dma] * _D,
        ],
    )


def _scatter_body(logp, place, zeros, out, idx4, valv, acc,
                  sem_i, sem_v, sem_s):
    cid = lax.axis_index("c")
    sid = lax.axis_index("s")

    @pl.when(cid == 0)
    def _():
        arow = _al(sid * _APT)
        pltpu.sync_copy(zeros.at[pl.ds(arow, _APT), :],
                        acc.at[pl.ds(arow, _APT), :])
        plsc.subcore_barrier()
        base = _al(sid * S_PT)
        iters = S_PT // CH_S

        def idx_spec(i, q):
            return [pltpu.make_async_copy(
                place.at[pl.ds(base + _al(i * CH_S), CH_S)],
                idx4.at[q], sem_i[q])]

        def val_spec(i, q, b):
            return [pltpu.make_async_copy(
                logp.at[pl.ds(base + _al(i * CH_S), CH_S), :],
                valv.at[b], sem_v[b])]

        def scat_spec(i, q, b):
            return [pltpu.make_async_copy(valv.at[b], acc.at[idx4.at[q]],
                                          sem_s[b])]

        _pipeline(iters, val_spec, scat_spec, idx_spec, n_gat=2, n_scat=2)
        plsc.subcore_barrier()
        pltpu.sync_copy(acc.at[pl.ds(arow, _APT), :],
                        out.at[pl.ds(arow, _APT), :])


@functools.cache
def _make_scatter():
    dma = pltpu.SemaphoreType.DMA
    return pl.kernel(
        _scatter_body,
        out_type=jax.ShapeDtypeStruct((NP, 8), jnp.float32),
        mesh=_mesh(),
        compiler_params=pltpu.CompilerParams(use_tc_tiling_on_sc=False),
        scratch_types=[
            pltpu.VMEM((_Q, CH_S), jnp.int32),
            pltpu.VMEM((_D, CH_S, 8), jnp.float32),
            pltpu.VMEM_SHARED((NP, 8), jnp.float32),
            [dma] * _Q,
            [dma] * _D,
            [dma] * _D,
        ],
    )


_R = 400


def _enc1_body(x_ref, a_ref, d_ref, w_ref, b_ref, h_ref, inv_ref):
    asum = a_ref[0] + a_ref[1]
    deg = jnp.maximum(d_ref[0][:, 0:1] + d_ref[1][:, 0:1], 1.0)
    inv = 1.0 / deg
    agg = asum * inv
    pre = jnp.dot(x_ref[...] + agg, w_ref[...],
                  preferred_element_type=jnp.float32) + b_ref[...]
    h_ref[...] = jnp.maximum(pre, 0.0)
    inv_ref[...] = jnp.broadcast_to(inv, (_R, 8))


_enc1 = pl.pallas_call(
    _enc1_body,
    grid=(N // _R,),
    in_specs=[
        pl.BlockSpec((_R, F), lambda i: (i, 0)),
        pl.BlockSpec((NC, _R, F), lambda i: (0, i, 0)),
        pl.BlockSpec((NC, _R, 8), lambda i: (0, i, 0)),
        pl.BlockSpec((F, F), lambda i: (0, 0)),
        pl.BlockSpec((1, F), lambda i: (0, 0)),
    ],
    out_specs=[
        pl.BlockSpec((_R, F), lambda i: (i, 0)),
        pl.BlockSpec((_R, 8), lambda i: (i, 0)),
    ],
    out_shape=[
        jax.ShapeDtypeStruct((N, F), jnp.float32),
        jax.ShapeDtypeStruct((N, 8), jnp.float32),
    ],
)


def _enc2_body(h_ref, a_ref, inv_ref, w_ref, b_ref, h2_ref):
    asum = a_ref[0] + a_ref[1]
    agg = asum * inv_ref[:, 0:1]
    pre = jnp.dot(h_ref[...] + agg, w_ref[...],
                  preferred_element_type=jnp.float32) + b_ref[...]
    h2_ref[...] = jnp.maximum(pre, 0.0)


_enc2 = pl.pallas_call(
    _enc2_body,
    grid=(N // _R,),
    in_specs=[
        pl.BlockSpec((_R, F), lambda i: (i, 0)),
        pl.BlockSpec((NC, _R, F), lambda i: (0, i, 0)),
        pl.BlockSpec((_R, 8), lambda i: (i, 0)),
        pl.BlockSpec((F, F), lambda i: (0, 0)),
        pl.BlockSpec((1, F), lambda i: (0, 0)),
    ],
    out_specs=pl.BlockSpec((_R, F), lambda i: (i, 0)),
    out_shape=jax.ShapeDtypeStruct((N, F), jnp.float32),
)


def _dec_body(f0_ref, f1_ref, f2_ref, w1_ref, b1_ref, w2_ref, b2_ref, o_ref):
    w = w1_ref[...]
    hid = (jnp.dot(f0_ref[...], w[0:128], preferred_element_type=jnp.float32)
           + jnp.dot(f1_ref[...], w[128:256],
                     preferred_element_type=jnp.float32)
           + jnp.dot(f2_ref[...], w[256:384],
                     preferred_element_type=jnp.float32)
           + b1_ref[...])
    hid = jnp.maximum(hid, 0.0)
    logits = jnp.dot(hid, w2_ref[...],
                     preferred_element_type=jnp.float32) + b2_ref[...]
    l0 = logits[:, 0:1]
    l1 = logits[:, 1:2]
    m = jnp.maximum(l0, l1)
    lse = m + jnp.log(jnp.exp(l0 - m) + jnp.exp(l1 - m))
    o_ref[...] = logits - lse


_RD = 2000
_PB = P // _RD
_dec = pl.pallas_call(
    _dec_body,
    grid=(_PB,),
    in_specs=[
        pl.BlockSpec((_RD, F), lambda i: (i, 0)),
        pl.BlockSpec((_RD, F), lambda i: (i + _PB, 0)),
        pl.BlockSpec((_RD, F), lambda i: (i + 2 * _PB, 0)),
        pl.BlockSpec((3 * F, 3 * F), lambda i: (0, 0)),
        pl.BlockSpec((1, 3 * F), lambda i: (0, 0)),
        pl.BlockSpec((3 * F, 8), lambda i: (0, 0)),
        pl.BlockSpec((1, 8), lambda i: (0, 0)),
    ],
    out_specs=pl.BlockSpec((_RD, 8), lambda i: (i, 0)),
    out_shape=jax.ShapeDtypeStruct((SP_TOT, 8), jnp.float32),
)


def kernel(x, edge_index, original, y, nodes, variants,
           W1, b1, W2, b2, Wd1, bd1, Wd2, bd2):
    f32 = jnp.float32
    i32 = jnp.int32
    src = edge_index[0]
    dst = edge_index[1]
    zeros = jnp.zeros((NP, F), f32)
    ones8 = jnp.ones((CH_E, 8), f32)

    parts1, deg1 = _make_seg(True)(x, src, dst, zeros, ones8)
    h, inv8 = _enc1(x, parts1, deg1, W1, b1.reshape(1, F))
    parts2 = _make_seg(False)(h, src, dst, zeros)
    h2 = _enc2(h, parts2, inv8, W2, b2.reshape(1, F))

    gpad = jnp.arange(GP_TOT - 3 * P, dtype=i32) % N
    gidx = jnp.concatenate([variants.reshape(3 * P), gpad])
    feats = _make_gather()(h2, gidx)

    Wd2p = jnp.concatenate([Wd2, jnp.zeros((3 * F, 8 - OUT), f32)], axis=1)
    bd2p = jnp.concatenate([bd2, jnp.zeros((8 - OUT,), f32)]).reshape(1, 8)
    logp8 = _dec(feats, feats, feats, Wd1,
                 bd1.reshape(1, 3 * F), Wd2p, bd2p)

    spad = N + jnp.arange(SP_TOT - P, dtype=i32) % (NP - N)
    placep = jnp.concatenate([variants[0], spad])
    pred8 = _make_scatter()(logp8, placep, jnp.zeros((NP, 8), f32))
    return pred8[:N, :OUT]


# speedup vs baseline: 1.5789x; 1.0181x over previous
import functools

import jax
import jax.numpy as jnp
from jax import lax
from jax.experimental import pallas as pl
from jax.experimental.pallas import tpu as pltpu
from jax.experimental.pallas import tpu_sc as plsc

N = 10000
E = 320000
P = 100000
F = 128
OUT = 2

NC = 2
NS = 16
NW = NC * NS

NP = 10240
_APT = NP // NS

CH_E = 80
CH_G = 128
CH_S = 80

E_CHT = E // (NW * CH_E)
G_CHT = 75
GP_TOT = NW * G_CHT * CH_G
S_PT = 6400
SP_TOT = S_PT * NS

_Q = 8
_D = 4


@functools.cache
def _mesh():
    return plsc.VectorSubcoreMesh(core_axis_name="c", subcore_axis_name="s",
                                  num_cores=NC, num_subcores=NS)


def _al(v):
    return pl.multiple_of(v, 8)


def _pipeline(iters, gather_spec, scat_spec, idx_spec, n_gat, n_scat,
              scat_add=True):
    lead = _Q - n_scat
    assert n_gat + n_scat <= _D and lead >= n_gat

    def start(descs, **kw):
        for d in descs:
            d.start(**kw)

    def wait(descs):
        for d in descs:
            d.wait()

    for i in range(lead):
        start(idx_spec(i, i % _Q))
    for i in range(n_gat):
        wait(idx_spec(i, i % _Q))
        start(gather_spec(i, i % _Q, i % _D))

    def step(i, j):
        q, b = j % _Q, j % _D
        wait(gather_spec(i, q, b))
        if scat_add:
            start(scat_spec(i, q, b), add=True)
        else:
            start(scat_spec(i, q, b))

        @pl.when(i >= n_scat)
        def _():
            wait(scat_spec(i - n_scat, (j - n_scat) % _Q, (j - n_scat) % _D))

        @pl.when(i + lead < iters)
        def _():
            start(idx_spec(i + lead, (j + lead) % _Q))

        @pl.when(i + n_gat < iters)
        def _():
            wait(idx_spec(i + n_gat, (j + n_gat) % _Q))
            start(gather_spec(i + n_gat, (j + n_gat) % _Q, (j + n_gat) % _D))

    def body(k, carry):
        i0 = _Q * k
        for j in range(_Q):
            step(i0 + j, j)
        return carry

    nb = iters // _Q
    lax.fori_loop(0, nb, body, 0)
    for j in range(iters - _Q * nb):
        step(jnp.int32(_Q * nb + j), j)
    for i in range(max(iters - n_scat, 0), iters):
        wait(scat_spec(jnp.int32(i), i % _Q, i % _D))


def _make_seg_body(with_deg):
    def _seg_body(table, src, dst, zeros, *rest):
        if with_deg:
            (ones, parts, partsd, idxs, idxd, rows, onesv, acc, accd,
             sem_i, sem_g, sem_s, sem_d) = rest
        else:
            parts, idxs, idxd, rows, acc, sem_i, sem_g, sem_s = rest
        cid = lax.axis_index("c")
        sid = lax.axis_index("s")
        arow = _al(sid * _APT)
        pltpu.sync_copy(zeros.at[pl.ds(arow, _APT), :],
                        acc.at[pl.ds(arow, _APT), :])
        if with_deg:
            pltpu.sync_copy(zeros.at[pl.ds(arow, _APT), 0:8],
                            accd.at[pl.ds(arow, _APT), :])
            pltpu.sync_copy(ones, onesv)
        plsc.subcore_barrier()
        wid = sid * NC + cid
        iters = E_CHT
        base = _al(wid * (E_CHT * CH_E))

        def idx_spec(i, q):
            return [
                pltpu.make_async_copy(src.at[pl.ds(base + _al(i * CH_E), CH_E)],
                                      idxs.at[q], sem_i[q]),
                pltpu.make_async_copy(dst.at[pl.ds(base + _al(i * CH_E), CH_E)],
                                      idxd.at[q], sem_i[q]),
            ]

        def gather_spec(i, q, b):
            return [pltpu.make_async_copy(table.at[idxs.at[q]], rows.at[b],
                                          sem_g[b])]

        def scat_spec(i, q, b):
            ds = [pltpu.make_async_copy(rows.at[b], acc.at[idxd.at[q]],
                                        sem_s[b])]
            if with_deg:
                ds.append(pltpu.make_async_copy(onesv, accd.at[idxd.at[q]],
                                                sem_d[b]))
            return ds

        _pipeline(iters, gather_spec, scat_spec, idx_spec, n_gat=2, n_scat=2)
        plsc.subcore_barrier()
        pltpu.sync_copy(acc.at[pl.ds(arow, _APT), :],
                        parts.at[cid, pl.ds(arow, _APT), :])
        if with_deg:
            pltpu.sync_copy(accd.at[pl.ds(arow, _APT), :],
                            partsd.at[cid, pl.ds(arow, _APT), :])

    return _seg_body


@functools.cache
def _make_seg(with_deg):
    dma = pltpu.SemaphoreType.DMA
    out_type = [jax.ShapeDtypeStruct((NC, NP, F), jnp.float32)]
    scratch = [
        pltpu.VMEM((_Q, CH_E), jnp.int32),
        pltpu.VMEM((_Q, CH_E), jnp.int32),
        pltpu.VMEM((_D, CH_E, F), jnp.float32),
    ]
    if with_deg:
        out_type.append(jax.ShapeDtypeStruct((NC, NP, 8), jnp.float32))
        scratch += [pltpu.VMEM((CH_E, 8), jnp.float32),
                    pltpu.VMEM_SHARED((NP, F), jnp.float32),
                    pltpu.VMEM_SHARED((NP, 8), jnp.float32),
                    [dma] * _Q, [dma] * _D, [dma] * _D, [dma] * _D]
    else:
        scratch += [pltpu.VMEM_SHARED((NP, F), jnp.float32),
                    [dma] * _Q, [dma] * _D, [dma] * _D]
    return pl.kernel(
        _make_seg_body(with_deg),
        out_type=tuple(out_type) if with_deg else out_type[0],
        mesh=_mesh(),
        compiler_params=pltpu.CompilerParams(use_tc_tiling_on_sc=False),
        scratch_types=scratch,
    )


def _gather_body(table, idx, out, idx4, rows, sem_i, sem_g, sem_w):
    cid = lax.axis_index("c")
    sid = lax.axis_index("s")
    wid = sid * NC + cid
    iters = G_CHT
    base = _al(wid * (G_CHT * CH_G))

    def idx_spec(i, q):
        return [pltpu.make_async_copy(idx.at[pl.ds(base + _al(i * CH_G), CH_G)],
                                      idx4.at[q], sem_i[q])]

    def gather_spec(i, q, b):
        return [pltpu.make_async_copy(table.at[idx4.at[q]], rows.at[b],
                                      sem_g[b])]

    def write_spec(i, q, b):
        return [pltpu.make_async_copy(
            rows.at[b], out.at[pl.ds(base + _al(i * CH_G), CH_G), :],
            sem_w[b])]

    _pipeline(iters, gather_spec, write_spec, idx_spec, n_gat=3, n_scat=1,
              scat_add=False)


@functools.cache
def _make_gather():
    dma = pltpu.SemaphoreType.DMA
    return pl.kernel(
        _gather_body,
        out_type=jax.ShapeDtypeStruct((GP_TOT, F), jnp.float32),
        mesh=_mesh(),
        compiler_params=pltpu.CompilerParams(use_tc_tiling_on_sc=False),
        scratch_types=[
            pltpu.VMEM((_Q, CH_G), jnp.int32),
            pltpu.VMEM((_D, CH_G, F), jnp.float32),
            [dma] * _Q,
            [dma] * _D,
            [dma] * _D,
        ],
    )


def _scatter_body(logp, place, zeros, out, idx4, valv, acc,
                  sem_i, sem_v, sem_s):
    cid = lax.axis_index("c")
    sid = lax.axis_index("s")

    @pl.when(cid == 0)
    def _():
        arow = _al(sid * _APT)
        pltpu.sync_copy(zeros.at[pl.ds(arow, _APT), :],
                        acc.at[pl.ds(arow, _APT), :])
        plsc.subcore_barrier()
        base = _al(sid * S_PT)
        iters = S_PT // CH_S

        def idx_spec(i, q):
            return [pltpu.make_async_copy(
                place.at[pl.ds(base + _al(i * CH_S), CH_S)],
                idx4.at[q], sem_i[q])]

        def val_spec(i, q, b):
            return [pltpu.make_async_copy(
                logp.at[pl.ds(base + _al(i * CH_S), CH_S), :],
                valv.at[b], sem_v[b])]

        def scat_spec(i, q, b):
            return [pltpu.make_async_copy(valv.at[b], acc.at[idx4.at[q]],
                                          sem_s[b])]

        _pipeline(iters, val_spec, scat_spec, idx_spec, n_gat=2, n_scat=2)
        plsc.subcore_barrier()
        pltpu.sync_copy(acc.at[pl.ds(arow, _APT), :],
                        out.at[pl.ds(arow, _APT), :])


@functools.cache
def _make_scatter():
    dma = pltpu.SemaphoreType.DMA
    return pl.kernel(
        _scatter_body,
        out_type=jax.ShapeDtypeStruct((NP, 8), jnp.float32),
        mesh=_mesh(),
        compiler_params=pltpu.CompilerParams(use_tc_tiling_on_sc=False),
        scratch_types=[
            pltpu.VMEM((_Q, CH_S), jnp.int32),
            pltpu.VMEM((_D, CH_S, 8), jnp.float32),
            pltpu.VMEM_SHARED((NP, 8), jnp.float32),
            [dma] * _Q,
            [dma] * _D,
            [dma] * _D,
        ],
    )


_R = 400


def _enc1_body(x_ref, a_ref, d_ref, w_ref, b_ref, h_ref, inv_ref):
    asum = a_ref[0] + a_ref[1]
    deg = jnp.maximum(d_ref[0][:, 0:1] + d_ref[1][:, 0:1], 1.0)
    inv = 1.0 / deg
    agg = asum * inv
    pre = jnp.dot(x_ref[...] + agg, w_ref[...],
                  preferred_element_type=jnp.float32) + b_ref[...]
    h_ref[...] = jnp.maximum(pre, 0.0)
    inv_ref[...] = jnp.broadcast_to(inv, (_R, 8))


_enc1 = pl.pallas_call(
    _enc1_body,
    grid=(N // _R,),
    in_specs=[
        pl.BlockSpec((_R, F), lambda i: (i, 0)),
        pl.BlockSpec((NC, _R, F), lambda i: (0, i, 0)),
        pl.BlockSpec((NC, _R, 8), lambda i: (0, i, 0)),
        pl.BlockSpec((F, F), lambda i: (0, 0)),
        pl.BlockSpec((1, F), lambda i: (0, 0)),
    ],
    out_specs=[
        pl.BlockSpec((_R, F), lambda i: (i, 0)),
        pl.BlockSpec((_R, 8), lambda i: (i, 0)),
    ],
    out_shape=[
        jax.ShapeDtypeStruct((N, F), jnp.float32),
        jax.ShapeDtypeStruct((N, 8), jnp.float32),
    ],
)


def _enc2_body(h_ref, a_ref, inv_ref, w_ref, b_ref, h2_ref):
    asum = a_ref[0] + a_ref[1]
    agg = asum * inv_ref[:, 0:1]
    pre = jnp.dot(h_ref[...] + agg, w_ref[...],
                  preferred_element_type=jnp.float32) + b_ref[...]
    h2_ref[...] = jnp.maximum(pre, 0.0)


_enc2 = pl.pallas_call(
    _enc2_body,
    grid=(N // _R,),
    in_specs=[
        pl.BlockSpec((_R, F), lambda i: (i, 0)),
        pl.BlockSpec((NC, _R, F), lambda i: (0, i, 0)),
        pl.BlockSpec((_R, 8), lambda i: (i, 0)),
        pl.BlockSpec((F, F), lambda i: (0, 0)),
        pl.BlockSpec((1, F), lambda i: (0, 0)),
    ],
    out_specs=pl.BlockSpec((_R, F), lambda i: (i, 0)),
    out_shape=jax.ShapeDtypeStruct((N, F), jnp.float32),
)


def _dec_body(f0_ref, f1_ref, f2_ref, w1_ref, b1_ref, w2_ref, b2_ref, o_ref):
    w = w1_ref[...]
    hid = (jnp.dot(f0_ref[...], w[0:128], preferred_element_type=jnp.float32)
           + jnp.dot(f1_ref[...], w[128:256],
                     preferred_element_type=jnp.float32)
           + jnp.dot(f2_ref[...], w[256:384],
                     preferred_element_type=jnp.float32)
           + b1_ref[...])
    hid = jnp.maximum(hid, 0.0)
    logits = jnp.dot(hid, w2_ref[...],
                     preferred_element_type=jnp.float32) + b2_ref[...]
    l0 = logits[:, 0:1]
    l1 = logits[:, 1:2]
    m = jnp.maximum(l0, l1)
    lse = m + jnp.log(jnp.exp(l0 - m) + jnp.exp(l1 - m))
    o_ref[...] = logits - lse


_RD = 4000
_PB = P // _RD
_dec = pl.pallas_call(
    _dec_body,
    grid=(_PB,),
    in_specs=[
        pl.BlockSpec((_RD, F), lambda i: (i, 0)),
        pl.BlockSpec((_RD, F), lambda i: (i + _PB, 0)),
        pl.BlockSpec((_RD, F), lambda i: (i + 2 * _PB, 0)),
        pl.BlockSpec((3 * F, 3 * F), lambda i: (0, 0)),
        pl.BlockSpec((1, 3 * F), lambda i: (0, 0)),
        pl.BlockSpec((3 * F, 8), lambda i: (0, 0)),
        pl.BlockSpec((1, 8), lambda i: (0, 0)),
    ],
    out_specs=pl.BlockSpec((_RD, 8), lambda i: (i, 0)),
    out_shape=jax.ShapeDtypeStruct((SP_TOT, 8), jnp.float32),
)


def kernel(x, edge_index, original, y, nodes, variants,
           W1, b1, W2, b2, Wd1, bd1, Wd2, bd2):
    f32 = jnp.float32
    i32 = jnp.int32
    src = edge_index[0]
    dst = edge_index[1]
    zeros = jnp.zeros((NP, F), f32)
    ones8 = jnp.ones((CH_E, 8), f32)

    parts1, deg1 = _make_seg(True)(x, src, dst, zeros, ones8)
    h, inv8 = _enc1(x, parts1, deg1, W1, b1.reshape(1, F))
    parts2 = _make_seg(False)(h, src, dst, zeros)
    h2 = _enc2(h, parts2, inv8, W2, b2.reshape(1, F))

    gpad = jnp.arange(GP_TOT - 3 * P, dtype=i32) % N
    gidx = jnp.concatenate([variants.reshape(3 * P), gpad])
    feats = _make_gather()(h2, gidx)

    Wd2p = jnp.concatenate([Wd2, jnp.zeros((3 * F, 8 - OUT), f32)], axis=1)
    bd2p = jnp.concatenate([bd2, jnp.zeros((8 - OUT,), f32)]).reshape(1, 8)
    logp8 = _dec(feats, feats, feats, Wd1,
                 bd1.reshape(1, 3 * F), Wd2p, bd2p)

    spad = N + jnp.arange(SP_TOT - P, dtype=i32) % (NP - N)
    placep = jnp.concatenate([variants[0], spad])
    pred8 = _make_scatter()(logp8, placep, jnp.zeros((NP, 8), f32))
    return pred8[:N, :OUT]
